# baseline (speedup 1.0000x reference)
import functools

import jax
import jax.numpy as jnp
from jax import lax
from jax.experimental import pallas as pl
from jax.experimental.pallas import tpu as pltpu
from jax.experimental.pallas import tpu_sc as plsc

_H = 32
_D = 5
_PAIRS = _D * 3
_EROWS = 1537 * _D
_TROWS = _EROWS + 512

_NC, _NS = 2, 16
_NW = _NC * _NS
_C = 128


def _table_body(e_ref, w_ref, o_ref):
    o_ref[...] = (jnp.dot(e_ref[...], w_ref[...],
                          preferred_element_type=jnp.float32)
                  * (1.0 / 3.0)).astype(jnp.bfloat16)


def _build_table(edge_w, wcat):
    return pl.pallas_call(
        _table_body,
        out_shape=jax.ShapeDtypeStruct((edge_w.shape[0], _D * _H),
                                       jnp.bfloat16),
    )(edge_w, wcat)


def _sc_body(table, idx, edge_out, spb_out,
             idx_v, acc_v, spb_v, et_v, st_v, sem_a, sem_s):
    wid = lax.axis_index("s") * _NC + lax.axis_index("c")
    n_chunks = idx.shape[1] // _NW
    iota2 = lax.iota(jnp.int32, 16) * 2
    zrow = jnp.zeros((2 * 16,), jnp.bfloat16)

    def sub(k, carry):
        c = wid * n_chunks + k
        sb = c // 32
        ll = c % 32
        pltpu.sync_copy(idx.at[:, c, :], idx_v)
        for r in range(_C):
            acc_v[r] = zrow
        cps = [pltpu.async_copy(table.at[idx_v.at[p]], acc_v, sem_a, add=True)
               for p in range(_PAIRS)]
        sp_cp = pltpu.async_copy(table.at[idx_v.at[_PAIRS]], spb_v, sem_s)
        for cp in cps:
            cp.wait()
        sp_cp.wait()
        for r in range(_C):
            lane = jnp.full((16,), r, jnp.int32)
            ev, od = plsc.unpack(acc_v[r], format=plsc.PackFormat.INTERLEAVED)
            plsc.store_scatter(et_v, [iota2, lane], ev)
            plsc.store_scatter(et_v, [iota2 + 1, lane], od)
            ev, od = plsc.unpack(spb_v[r], format=plsc.PackFormat.INTERLEAVED)
            plsc.store_scatter(st_v, [iota2, lane], ev)
            plsc.store_scatter(st_v, [iota2 + 1, lane], od)
        pltpu.sync_copy(et_v, edge_out.at[pl.ds(sb * _H, _H), ll, :])
        pltpu.sync_copy(st_v, spb_out.at[pl.ds(sb * _H, _H), ll, :])
        return carry

    lax.fori_loop(0, n_chunks, sub, 0)


def _sc_gather(table, idx, b):
    out_sh = jax.ShapeDtypeStruct((b * _H, 32, 128), jnp.float32)
    f = functools.partial(
        pl.kernel,
        out_type=[out_sh, out_sh],
        mesh=plsc.VectorSubcoreMesh(core_axis_name="c", subcore_axis_name="s",
                                    num_cores=_NC, num_subcores=_NS),
        scratch_types=[pltpu.VMEM((_PAIRS + 1, _C), jnp.int32),
                       pltpu.VMEM((_C, _H), jnp.bfloat16),
                       pltpu.VMEM((_C, _H), jnp.bfloat16),
                       pltpu.VMEM((_H, _C), jnp.float32),
                       pltpu.VMEM((_H, _C), jnp.float32),
                       pltpu.SemaphoreType.DMA,
                       pltpu.SemaphoreType.DMA],
        compiler_params=pltpu.CompilerParams(use_tc_tiling_on_sc=False,
                                             needs_layout_passes=False),
    )(_sc_body)
    return f(table, idx)


def _asm_body(ab_ref, sp_ref, edge_ref, spb_ref, t_ref, o_ref):
    n = 64
    sp = sp_ref[0]
    spc = jnp.where(sp == 0, 1, sp)
    spc = jnp.where(spc > 1, spc - 1, spc)
    spc = jnp.clip(spc, 0, _D)
    rs = 1.0 / spc.astype(jnp.float32)
    interior = (edge_ref[...] * rs[None]
                + spb_ref[...]
                + 2.0 * ab_ref[0][None])
    t = t_ref[...]
    top = jnp.concatenate(
        [jnp.zeros((_H, 1), jnp.float32), jnp.broadcast_to(t, (_H, n))],
        axis=1)
    o_ref[0, :, 0, :] = top
    for i in range(n):
        seg = interior[:, i // 2, (i % 2) * n:(i % 2) * n + n]
        o_ref[0, :, i + 1, :] = jnp.concatenate([t, seg], axis=1)


def _assemble(ab3, sp3, edge_t, spb_t, t_col):
    b = ab3.shape[0]
    n1 = 65
    return pl.pallas_call(
        _asm_body,
        grid=(b,),
        in_specs=[
            pl.BlockSpec((1, 32, 128), lambda i: (i, 0, 0)),
            pl.BlockSpec((1, 32, 128), lambda i: (i, 0, 0)),
            pl.BlockSpec((_H, 32, 128), lambda i: (i, 0, 0)),
            pl.BlockSpec((_H, 32, 128), lambda i: (i, 0, 0)),
            pl.BlockSpec((_H, 1), lambda i: (0, 0)),
        ],
        out_specs=pl.BlockSpec((1, _H, n1, n1), lambda i: (i, 0, 0, 0)),
        out_shape=jax.ShapeDtypeStruct((b, _H, n1, n1), jnp.float32),
    )(ab3, sp3, edge_t, spb_t, t_col)


def kernel(attn_bias, spatial_pos, x, edge_input, attn_edge_type,
           edge_encoder_w, spatial_pos_encoder_w, edge_dis_encoder_w,
           graph_token_virtual_distance_w):
    b, n = x.shape[0], x.shape[1]
    bnn = b * n * n

    w = edge_dis_encoder_w.reshape(-1, _H, _H)[:_D]
    wcat = jnp.transpose(w, (1, 0, 2)).reshape(_H, _D * _H)
    ct = _build_table(edge_encoder_w, wcat)
    table = jnp.concatenate(
        [ct.reshape(_EROWS, _H),
         spatial_pos_encoder_w.astype(jnp.bfloat16)], axis=0)

    e = edge_input.astype(jnp.int32)
    eidx = e * _D + jnp.arange(_D, dtype=jnp.int32)[:, None]
    eidx = eidx.reshape(bnn, _PAIRS)
    sidx = spatial_pos.astype(jnp.int32).reshape(bnn, 1) + _EROWS
    idx = jnp.concatenate([eidx, sidx], axis=1).T.reshape(_PAIRS + 1,
                                                          bnn // _C, _C)

    edge_t, spb_t = _sc_gather(table, idx, b)

    return _assemble(
        attn_bias.reshape(b, 32, 128),
        spatial_pos.astype(jnp.int32).reshape(b, 32, 128),
        edge_t, spb_t,
        graph_token_virtual_distance_w.reshape(_H, 1),
    )
